# baseline (speedup 1.0000x reference)
import jax
import jax.numpy as jnp
import numpy as np
from jax.experimental import pallas as pl
from jax.experimental.pallas import tpu as pltpu

_IDX = (3, 7, 15, 22, 31, 44, 58, 63, 71, 85, 92, 101, 110, 118, 124, 127)

_SEL_T = np.zeros((16, 128), dtype=np.float32)
for _k, _i in enumerate(_IDX):
    _SEL_T[_k, _i] = 1.0

_NF = 200


_DEPTH = 8
_HALF = _NF // 2


def _gather_body(x_hbm, s_ref, o_ref, xs_ref, sem):
    h = pl.program_id(0)
    i = pl.program_id(1)
    f = h * _HALF + i

    @pl.when(i == 0)
    def _first():
        for d in range(_DEPTH - 1):
            pltpu.make_async_copy(
                x_hbm.at[:, f + d, :], xs_ref.at[d], sem.at[d]).start()

    @pl.when(i + _DEPTH - 1 < _HALF)
    def _prefetch():
        nxt = i + _DEPTH - 1
        pltpu.make_async_copy(
            x_hbm.at[:, f + _DEPTH - 1, :], xs_ref.at[nxt % _DEPTH],
            sem.at[nxt % _DEPTH]).start()

    pltpu.make_async_copy(
        x_hbm.at[:, f, :], xs_ref.at[i % _DEPTH], sem.at[i % _DEPTH]).wait()
    x = xs_ref[i % _DEPTH]
    g_t = jax.lax.dot_general(
        s_ref[...], x, (((1,), (1,)), ((), ())),
        preferred_element_type=jnp.float32)
    o_ref[...] = g_t.reshape(o_ref.shape)


def kernel(inputs):
    n = inputs.shape[0]
    sel_t = jnp.asarray(_SEL_T)
    out_t = pl.pallas_call(
        _gather_body,
        grid=(2, _HALF),
        in_specs=[
            pl.BlockSpec(memory_space=pl.ANY),
            pl.BlockSpec((16, 128), lambda h, i: (0, 0)),
        ],
        out_specs=pl.BlockSpec((1, 16, n), lambda h, i: (h * _HALF + i, 0, 0)),
        out_shape=jax.ShapeDtypeStruct((200, 16, n), inputs.dtype),
        scratch_shapes=[
            pltpu.VMEM((_DEPTH, n, 128), jnp.float32),
            pltpu.SemaphoreType.DMA((_DEPTH,)),
        ],
        compiler_params=pltpu.CompilerParams(
            dimension_semantics=("parallel", "arbitrary")),
    )(inputs, sel_t)
    return out_t.transpose(2, 0, 1)
